# baseline (speedup 1.0000x reference)
import functools

import jax
import jax.numpy as jnp
from jax import lax
from jax.experimental import pallas as pl
from jax.experimental.pallas import tpu as pltpu
from jax.experimental.pallas import tpu_sc as plsc

N_X = 5736
N_GENE = 4264
N_TOTAL = N_X + N_GENE
E = 48000
NUM_REL = 4

MP = 10240
K1P = 1664
D1 = 1600
D1P = 1664
D2P = 1024
HW1 = 512
HW2 = 128

E_PAD = 49152
CB = 32
CNT_SLOTS = 49152

_MESH = dict(core_axis_name="c", subcore_axis_name="s")


def _lane(vec, i):
    return jnp.sum(jnp.where(lax.iota(jnp.int32, 16) == i, vec, 0))


def _sc_scale_body(gc_hbm, scale_hbm, cnt_v, gblk_v, sblk_v):
    c = lax.axis_index("c")
    s = lax.axis_index("s")
    ones = jnp.ones((16,), jnp.float32)

    @pl.when(jnp.logical_and(c == 0, s == 0))
    def _():
        def zero_body(i, _):
            cnt_v[pl.ds(i * 16, 16)] = jnp.zeros((16,), jnp.float32)
            return 0
        lax.fori_loop(0, CNT_SLOTS // 16, zero_body, 0)

        def count_blk(blk, _):
            pltpu.sync_copy(gc_hbm.at[pl.ds(blk * 1024, 1024)], gblk_v)

            def count_in(k, _):
                idx = gblk_v[pl.ds(k * 16, 16)]
                plsc.addupdate_scatter(cnt_v, [idx], ones)
                return 0
            lax.fori_loop(0, 64, count_in, 0)
            return 0
        lax.fori_loop(0, E_PAD // 1024, count_blk, 0)

        def scale_blk(blk, _):
            pltpu.sync_copy(gc_hbm.at[pl.ds(blk * 1024, 1024)], gblk_v)

            def scale_in(k, _):
                idx = gblk_v[pl.ds(k * 16, 16)]
                vals = plsc.load_gather(cnt_v, [idx])
                sblk_v[pl.ds(k * 16, 16)] = 1.0 / jnp.maximum(vals, 1.0)
                return 0
            lax.fori_loop(0, 64, scale_in, 0)
            pltpu.sync_copy(sblk_v, scale_hbm.at[pl.ds(blk * 1024, 1024)])
            return 0
        lax.fori_loop(0, E_PAD // 1024, scale_blk, 0)


def _sc_scale(g_cnt):
    kern = pl.kernel(
        _sc_scale_body,
        out_type=jax.ShapeDtypeStruct((E_PAD,), jnp.float32),
        mesh=plsc.VectorSubcoreMesh(**_MESH),
        compiler_params=pltpu.CompilerParams(needs_layout_passes=False),
        scratch_types=[
            pltpu.VMEM((CNT_SLOTS,), jnp.float32),
            pltpu.VMEM((1024,), jnp.int32),
            pltpu.VMEM((1024,), jnp.float32),
        ],
    )
    return kern(g_cnt)


NBT = 32
NBLK = MP // NBT


def _sc_msg_body(dpad, nja, dw, ym_hbm, yr_hbm, g_hbm, eds_hbm, meta_hbm, bias_hbm, out_hbm,
                 meta_v, bias_v, eds_v, idxa_v, idxb_v, rows_v, acc_v,
                 y8_v, gsem0, gsem1):
    c = lax.axis_index("c")
    s = lax.axis_index("s")
    w = c * 16 + s
    iota = lax.iota(jnp.int32, 16)
    nj = dpad // 16
    njw = dw // 16

    pltpu.sync_copy(meta_hbm, meta_v)
    pltpu.sync_copy(bias_hbm, bias_v)

    def z0(i, _):
        acc_v[pl.ds(i * 16, 16)] = jnp.zeros((16,), jnp.float32)
        return 0
    lax.fori_loop(0, (NBT + 1) * nj, z0, 0)

    gsems = (gsem0, gsem1)
    idxs = (idxa_v, idxb_v)

    def block_body(k, _):
        bid = k * 32 + w
        cb = pl.multiple_of(bid // 16 * 16, 16)
        ln = bid - cb
        lo8 = _lane(meta_v[pl.ds(cb, 16)], ln)
        nwin = _lane(meta_v[pl.ds(NBLK + cb, 16)], ln)
        lot = _lane(meta_v[pl.ds(2 * NBLK + cb, 16)], ln)
        hit = _lane(meta_v[pl.ds(3 * NBLK + cb, 16)], ln)
        bnb = bid * NBT

        def win_body(wi, _):
            wbase = pl.multiple_of(lo8 + wi * 128, 8)
            eoff = pl.multiple_of(wbase * 2, 8)
            pltpu.sync_copy(eds_hbm.at[pl.ds(eoff, 256)], eds_v)

            def issue(sub, par):
                sb8 = pl.multiple_of(wbase + sub * 16, 8)
                pltpu.sync_copy(g_hbm.at[pl.ds(sb8, 16)], idxs[par])
                pltpu.async_copy(ym_hbm.at[idxs[par]],
                                 rows_v.at[par], gsems[par])

            def wait(sub, par):
                pltpu.make_async_copy(ym_hbm.at[idxs[par]],
                                      rows_v.at[par], gsems[par]).wait()

            def process(sub, par):
                sbase = wbase + sub * 16
                evec = sbase + iota
                dl = eds_v[pl.ds(sub * 32, 16)] - bnb
                inr = ((evec >= lot) & (evec < hit)
                       & (dl >= 0) & (dl < NBT))
                dlc = jnp.where(inr, dl, NBT)
                sv = jnp.where(
                    inr, plsc.bitcast(eds_v[pl.ds(sub * 32 + 16, 16)],
                                      jnp.float32),
                    jnp.zeros((16,), jnp.float32))
                sscs = [sv[e] for e in range(16)]
                abss = [dlc[e] * dpad for e in range(16)]

                def jcol(j, _, par=par):
                    off = j * 16
                    for e in range(16):
                        sl = pl.ds(abss[e] + off, 16)
                        acc_v[sl] = (acc_v[sl]
                                     + rows_v[par, e, pl.ds(off, 16)]
                                     * sscs[e])
                    return 0
                lax.fori_loop(0, nj, jcol, 0)

            @pl.when(wbase < hit)
            def _():
                issue(0, 0)
            for sub in range(8):
                par = sub & 1
                if sub + 1 < 8:
                    @pl.when(wbase + (sub + 1) * 16 < hit)
                    def _(sub=sub, par=par):
                        issue(sub + 1, 1 - par)

                @pl.when(wbase + sub * 16 < hit)
                def _(sub=sub, par=par):
                    wait(sub, par)
                    process(sub, par)
            return 0
        lax.fori_loop(0, nwin, win_body, 0)

        def drain_body(r8, _):
            grow = pl.multiple_of(bnb + r8 * 8, 8)
            pltpu.sync_copy(yr_hbm.at[pl.ds(grow, 8)], y8_v)
            for rr in range(8):
                def db(j, _, rr=rr):
                    sl = pl.ds(j * 16, 16)
                    aoff = pl.ds((r8 * 8 + rr) * dpad + j * 16, 16)
                    y8_v[rr, sl] = jnp.maximum(
                        acc_v[aoff] + y8_v[rr, sl] + bias_v[sl], 0.0)
                    acc_v[aoff] = jnp.zeros((16,), jnp.float32)
                    return 0
                lax.fori_loop(0, njw, db, 0, unroll=4)
            pltpu.sync_copy(y8_v, out_hbm.at[pl.ds(grow, 8)])
            return 0
        lax.fori_loop(0, NBT // 8, drain_body, 0)
        return 0
    lax.fori_loop(0, NBLK // 32, block_body, 0)


def _sc_msg(ymsg, yroot, g_y, eds, meta, bias, dpad, nja, dw):
    kern = pl.kernel(
        functools.partial(_sc_msg_body, dpad, nja, dw),
        out_type=jax.ShapeDtypeStruct((MP, dw), jnp.float32),
        mesh=plsc.VectorSubcoreMesh(**_MESH),
        compiler_params=pltpu.CompilerParams(needs_layout_passes=False),
        scratch_types=[
            pltpu.VMEM((4 * NBLK,), jnp.int32),
            pltpu.VMEM((dw,), jnp.float32),
            pltpu.VMEM((256,), jnp.int32),
            pltpu.VMEM((16,), jnp.int32),
            pltpu.VMEM((16,), jnp.int32),
            pltpu.VMEM((2, 16, dpad), jnp.float32),
            pltpu.VMEM(((NBT + 1) * dpad,), jnp.float32),
            pltpu.VMEM((8, dw), jnp.float32),
            pltpu.SemaphoreType.DMA,
            pltpu.SemaphoreType.DMA,
        ],
    )
    return kern(ymsg, yroot, g_y, eds, meta, bias)


def _mm_msg_body(dpad, x_ref, w_ref, o_ref):
    d = jnp.dot(x_ref[...], w_ref[0], preferred_element_type=jnp.float32)
    pad = dpad - d.shape[1]
    if pad:
        d = jnp.concatenate(
            [d, jnp.zeros((d.shape[0], pad), jnp.float32)], axis=1)
    o_ref[0] = d


def _mm_msg(x, w, dpad, bm=256):
    r, k, dout = w.shape
    return pl.pallas_call(
        functools.partial(_mm_msg_body, dpad),
        grid=(r, MP // bm),
        in_specs=[
            pl.BlockSpec((bm, k), lambda ri, mi: (mi, 0)),
            pl.BlockSpec((1, k, dout), lambda ri, mi: (ri, 0, 0)),
        ],
        out_specs=pl.BlockSpec((1, bm, dpad), lambda ri, mi: (ri, mi, 0)),
        out_shape=jax.ShapeDtypeStruct((r, MP, dpad), jnp.float32),
    )(x, w)


def _mm_root_body(x_ref, w_ref, o_ref):
    o_ref[...] = jnp.dot(x_ref[...], w_ref[...],
                         preferred_element_type=jnp.float32)


def _mm_root(x, w, bm=256):
    k, dout = w.shape
    return pl.pallas_call(
        _mm_root_body,
        grid=(MP // bm,),
        in_specs=[
            pl.BlockSpec((bm, k), lambda mi: (mi, 0)),
            pl.BlockSpec((k, dout), lambda mi: (0, 0)),
        ],
        out_specs=pl.BlockSpec((bm, dout), lambda mi: (mi, 0)),
        out_shape=jax.ShapeDtypeStruct((MP, dout), jnp.float32),
    )(x, w)


def _head_body(x_ref, w1_ref, b1_ref, w2_ref, b2_ref, o_ref):
    a = jnp.dot(x_ref[...], w1_ref[...], preferred_element_type=jnp.float32)
    a = jnp.maximum(a + b1_ref[0:1, :], 0.0)
    z = jnp.dot(a, w2_ref[...], preferred_element_type=jnp.float32)
    z = z + b2_ref[0:1, :]
    z0 = z[:, 0:1]
    z1 = z[:, 1:2]
    m = jnp.maximum(z0, z1)
    lse = m + jnp.log(jnp.exp(z0 - m) + jnp.exp(z1 - m))
    o_ref[...] = z - lse


def _head(x, w1, b1, w2, b2, bm=256):
    grid = (MP // bm,)
    return pl.pallas_call(
        _head_body,
        grid=grid,
        in_specs=[
            pl.BlockSpec((bm, D2P), lambda mi: (mi, 0)),
            pl.BlockSpec((D2P, HW1), lambda mi: (0, 0)),
            pl.BlockSpec((8, HW1), lambda mi: (0, 0)),
            pl.BlockSpec((HW1, HW2), lambda mi: (0, 0)),
            pl.BlockSpec((8, HW2), lambda mi: (0, 0)),
        ],
        out_specs=pl.BlockSpec((bm, HW2), lambda mi: (mi, 0)),
        out_shape=jax.ShapeDtypeStruct((MP, HW2), jnp.float32),
    )(x, w1, b1, w2, b2)


def _edge_meta(dst_s):
    bstart = jnp.searchsorted(
        dst_s, jnp.arange(NBLK + 1, dtype=jnp.int32) * NBT).astype(jnp.int32)
    lot = bstart[:-1]
    hit = bstart[1:]
    lo16 = lot // 16 * 16
    nwin = (hit - lo16 + 127) // 128
    return jnp.concatenate([lo16, nwin, lot, hit]).astype(jnp.int32)


def kernel(x, edge_index, edge_type, gene_emb, W1, root1, b1, W2, root2, b2,
           lin1_w, lin1_b, lin2_w, lin2_b):
    f32 = jnp.float32

    h1 = jnp.zeros((MP, 1613), f32)
    h1 = h1.at[:N_X].set(x).at[N_X:N_TOTAL].set(gene_emb)

    root2p = jnp.zeros((D1, D2P), f32).at[:, :900].set(root2)
    b2p = jnp.zeros((D2P,), f32).at[:900].set(b2)

    l1w = jnp.zeros((D2P, HW1), f32).at[:900, :400].set(lin1_w)
    l1b = jnp.zeros((8, HW1), f32).at[0, :400].set(lin1_b)
    l2w = jnp.zeros((HW1, HW2), f32).at[:400, :2].set(lin2_w)
    l2b = jnp.zeros((8, HW2), f32).at[0, :2].set(lin2_b)

    src = edge_index[0].astype(jnp.int32)
    dst = edge_index[1].astype(jnp.int32)
    et = edge_type.astype(jnp.int32)
    order = jnp.argsort(dst)
    npad = E_PAD - E
    src_s = jnp.concatenate([src[order], jnp.zeros((npad,), jnp.int32)])
    dst_s = jnp.concatenate([dst[order], jnp.full((npad,), MP, jnp.int32)])
    et_s = jnp.concatenate([et[order], jnp.zeros((npad,), jnp.int32)])

    g_y = et_s * MP + src_s
    g_c = jnp.where(jnp.arange(E_PAD) < E,
                    et_s * N_TOTAL + dst_s, CNT_SLOTS - 1)
    g_c = g_c.astype(jnp.int32)

    meta = _edge_meta(dst_s)

    scale = _sc_scale(g_c)
    scale_bits = lax.bitcast_convert_type(scale, jnp.int32)
    eds = jnp.stack([dst_s.reshape(-1, 16),
                     scale_bits.reshape(-1, 16)], axis=1).reshape(-1)

    y1m = _mm_msg(h1, W1, D1P).reshape(4 * MP, D1P)
    y1r = _mm_root(h1, root1)
    h2 = _sc_msg(y1m, y1r, g_y, eds, meta, b1, D1P, D1 // 16, D1)

    y2m = _mm_msg(h2, W2, D2P).reshape(4 * MP, D2P)
    y2r = _mm_root(h2, root2p)
    h3 = _sc_msg(y2m, y2r, g_y, eds, meta, b2p, D2P, 57, D2P)

    out = _head(h3, l1w, l1b, l2w, l2b)
    return out[:N_TOTAL, :2]


# speedup vs baseline: 1.1004x; 1.1004x over previous
import functools

import jax
import jax.numpy as jnp
from jax import lax
from jax.experimental import pallas as pl
from jax.experimental.pallas import tpu as pltpu
from jax.experimental.pallas import tpu_sc as plsc

N_X = 5736
N_GENE = 4264
N_TOTAL = N_X + N_GENE
E = 48000
NUM_REL = 4

MP = 10240
K1P = 1664
D1 = 1600
D1P = 1664
D2P = 1024
HW1 = 512
HW2 = 128

E_PAD = 49152
CB = 32
CNT_SLOTS = 49152

_MESH = dict(core_axis_name="c", subcore_axis_name="s")


def _lane(vec, i):
    return jnp.sum(jnp.where(lax.iota(jnp.int32, 16) == i, vec, 0))


def _sc_scale_body(gc_hbm, scale_hbm, cnt_v, gblk_v, sblk_v):
    c = lax.axis_index("c")
    s = lax.axis_index("s")
    ones = jnp.ones((16,), jnp.float32)

    @pl.when(jnp.logical_and(c == 0, s == 0))
    def _():
        def zero_body(i, _):
            cnt_v[pl.ds(i * 16, 16)] = jnp.zeros((16,), jnp.float32)
            return 0
        lax.fori_loop(0, CNT_SLOTS // 16, zero_body, 0)

        def count_blk(blk, _):
            pltpu.sync_copy(gc_hbm.at[pl.ds(blk * 1024, 1024)], gblk_v)

            def count_in(k, _):
                idx = gblk_v[pl.ds(k * 16, 16)]
                plsc.addupdate_scatter(cnt_v, [idx], ones)
                return 0
            lax.fori_loop(0, 64, count_in, 0)
            return 0
        lax.fori_loop(0, E_PAD // 1024, count_blk, 0)

        def scale_blk(blk, _):
            pltpu.sync_copy(gc_hbm.at[pl.ds(blk * 1024, 1024)], gblk_v)

            def scale_in(k, _):
                idx = gblk_v[pl.ds(k * 16, 16)]
                vals = plsc.load_gather(cnt_v, [idx])
                sblk_v[pl.ds(k * 16, 16)] = 1.0 / jnp.maximum(vals, 1.0)
                return 0
            lax.fori_loop(0, 64, scale_in, 0)
            pltpu.sync_copy(sblk_v, scale_hbm.at[pl.ds(blk * 1024, 1024)])
            return 0
        lax.fori_loop(0, E_PAD // 1024, scale_blk, 0)


def _sc_scale(g_cnt):
    kern = pl.kernel(
        _sc_scale_body,
        out_type=jax.ShapeDtypeStruct((E_PAD,), jnp.float32),
        mesh=plsc.VectorSubcoreMesh(**_MESH),
        compiler_params=pltpu.CompilerParams(needs_layout_passes=False),
        scratch_types=[
            pltpu.VMEM((CNT_SLOTS,), jnp.float32),
            pltpu.VMEM((1024,), jnp.int32),
            pltpu.VMEM((1024,), jnp.float32),
        ],
    )
    return kern(g_cnt)


NBT = 32
NBLK = MP // NBT


def _sc_msg_body(dpad, nja, dw, ym_hbm, yr_hbm, g_hbm, eds_hbm, meta_hbm, bias_hbm, out_hbm,
                 meta_v, bias_v, eds_v, idxa_v, idxb_v, rows_v, acc_v,
                 y8_v, gsem0, gsem1):
    c = lax.axis_index("c")
    s = lax.axis_index("s")
    w = c * 16 + s
    iota = lax.iota(jnp.int32, 16)
    nj = dpad // 16
    njw = dw // 16

    pltpu.sync_copy(meta_hbm, meta_v)
    pltpu.sync_copy(bias_hbm, bias_v)

    def z0(i, _):
        acc_v[pl.ds(i * 16, 16)] = jnp.zeros((16,), jnp.float32)
        return 0
    lax.fori_loop(0, (NBT + 1) * nj, z0, 0)

    gsems = (gsem0, gsem1)
    idxs = (idxa_v, idxb_v)

    def block_body(k, _):
        bid = k * 32 + w
        cb = pl.multiple_of(bid // 16 * 16, 16)
        ln = bid - cb
        lo8 = _lane(meta_v[pl.ds(cb, 16)], ln)
        nwin = _lane(meta_v[pl.ds(NBLK + cb, 16)], ln)
        lot = _lane(meta_v[pl.ds(2 * NBLK + cb, 16)], ln)
        hit = _lane(meta_v[pl.ds(3 * NBLK + cb, 16)], ln)
        bnb = bid * NBT

        def win_body(wi, _):
            wbase = pl.multiple_of(lo8 + wi * 128, 8)
            eoff = pl.multiple_of(wbase * 2, 8)
            pltpu.sync_copy(eds_hbm.at[pl.ds(eoff, 256)], eds_v)

            def issue(sub, par):
                sb8 = pl.multiple_of(wbase + sub * 16, 8)
                pltpu.sync_copy(g_hbm.at[pl.ds(sb8, 16)], idxs[par])
                pltpu.async_copy(ym_hbm.at[idxs[par]],
                                 rows_v.at[par], gsems[par])

            def wait(sub, par):
                pltpu.make_async_copy(ym_hbm.at[idxs[par]],
                                      rows_v.at[par], gsems[par]).wait()

            def process(sub, par):
                sbase = wbase + sub * 16
                evec = sbase + iota
                dl = eds_v[pl.ds(sub * 32, 16)] - bnb
                inr = ((evec >= lot) & (evec < hit)
                       & (dl >= 0) & (dl < NBT))
                dlc = jnp.where(inr, dl, NBT)
                sv = jnp.where(
                    inr, plsc.bitcast(eds_v[pl.ds(sub * 32 + 16, 16)],
                                      jnp.float32),
                    jnp.zeros((16,), jnp.float32))
                sscs = [sv[e] for e in range(16)]
                abss = [dlc[e] * dpad for e in range(16)]

                def jcol(j, _, par=par):
                    off = j * 16
                    for e in range(16):
                        plsc.addupdate(
                            acc_v.at[pl.ds(abss[e] + off, 16)],
                            rows_v[par, e, pl.ds(off, 16)] * sscs[e])
                    return 0
                lax.fori_loop(0, nj, jcol, 0)

            @pl.when(wbase < hit)
            def _():
                issue(0, 0)
            for sub in range(8):
                par = sub & 1
                if sub + 1 < 8:
                    @pl.when(wbase + (sub + 1) * 16 < hit)
                    def _(sub=sub, par=par):
                        issue(sub + 1, 1 - par)

                @pl.when(wbase + sub * 16 < hit)
                def _(sub=sub, par=par):
                    wait(sub, par)
                    process(sub, par)
            return 0
        lax.fori_loop(0, nwin, win_body, 0)

        def drain_body(r8, _):
            grow = pl.multiple_of(bnb + r8 * 8, 8)
            pltpu.sync_copy(yr_hbm.at[pl.ds(grow, 8)], y8_v)
            for rr in range(8):
                def db(j, _, rr=rr):
                    sl = pl.ds(j * 16, 16)
                    aoff = pl.ds((r8 * 8 + rr) * dpad + j * 16, 16)
                    y8_v[rr, sl] = jnp.maximum(
                        acc_v[aoff] + y8_v[rr, sl] + bias_v[sl], 0.0)
                    acc_v[aoff] = jnp.zeros((16,), jnp.float32)
                    return 0
                lax.fori_loop(0, njw, db, 0, unroll=4)
            pltpu.sync_copy(y8_v, out_hbm.at[pl.ds(grow, 8)])
            return 0
        lax.fori_loop(0, NBT // 8, drain_body, 0)
        return 0
    lax.fori_loop(0, NBLK // 32, block_body, 0)


def _sc_msg(ymsg, yroot, g_y, eds, meta, bias, dpad, nja, dw):
    kern = pl.kernel(
        functools.partial(_sc_msg_body, dpad, nja, dw),
        out_type=jax.ShapeDtypeStruct((MP, dw), jnp.float32),
        mesh=plsc.VectorSubcoreMesh(**_MESH),
        compiler_params=pltpu.CompilerParams(needs_layout_passes=False),
        scratch_types=[
            pltpu.VMEM((4 * NBLK,), jnp.int32),
            pltpu.VMEM((dw,), jnp.float32),
            pltpu.VMEM((256,), jnp.int32),
            pltpu.VMEM((16,), jnp.int32),
            pltpu.VMEM((16,), jnp.int32),
            pltpu.VMEM((2, 16, dpad), jnp.float32),
            pltpu.VMEM(((NBT + 1) * dpad,), jnp.float32),
            pltpu.VMEM((8, dw), jnp.float32),
            pltpu.SemaphoreType.DMA,
            pltpu.SemaphoreType.DMA,
        ],
    )
    return kern(ymsg, yroot, g_y, eds, meta, bias)


def _mm_msg_body(dpad, x_ref, w_ref, o_ref):
    d = jnp.dot(x_ref[...].astype(jnp.bfloat16), w_ref[0].astype(jnp.bfloat16),
                preferred_element_type=jnp.float32)
    pad = dpad - d.shape[1]
    if pad:
        d = jnp.concatenate(
            [d, jnp.zeros((d.shape[0], pad), jnp.float32)], axis=1)
    o_ref[0] = d


def _mm_msg(x, w, dpad, bm=256):
    r, k, dout = w.shape
    return pl.pallas_call(
        functools.partial(_mm_msg_body, dpad),
        grid=(r, MP // bm),
        in_specs=[
            pl.BlockSpec((bm, k), lambda ri, mi: (mi, 0)),
            pl.BlockSpec((1, k, dout), lambda ri, mi: (ri, 0, 0)),
        ],
        out_specs=pl.BlockSpec((1, bm, dpad), lambda ri, mi: (ri, mi, 0)),
        out_shape=jax.ShapeDtypeStruct((r, MP, dpad), jnp.float32),
    )(x, w)


def _mm_root_body(x_ref, w_ref, o_ref):
    o_ref[...] = jnp.dot(x_ref[...].astype(jnp.bfloat16),
                         w_ref[...].astype(jnp.bfloat16),
                         preferred_element_type=jnp.float32)


def _mm_root(x, w, bm=256):
    k, dout = w.shape
    return pl.pallas_call(
        _mm_root_body,
        grid=(MP // bm,),
        in_specs=[
            pl.BlockSpec((bm, k), lambda mi: (mi, 0)),
            pl.BlockSpec((k, dout), lambda mi: (0, 0)),
        ],
        out_specs=pl.BlockSpec((bm, dout), lambda mi: (mi, 0)),
        out_shape=jax.ShapeDtypeStruct((MP, dout), jnp.float32),
    )(x, w)


def _head_body(x_ref, w1_ref, b1_ref, w2_ref, b2_ref, o_ref):
    a = jnp.dot(x_ref[...], w1_ref[...], preferred_element_type=jnp.float32)
    a = jnp.maximum(a + b1_ref[0:1, :], 0.0)
    z = jnp.dot(a, w2_ref[...], preferred_element_type=jnp.float32)
    z = z + b2_ref[0:1, :]
    z0 = z[:, 0:1]
    z1 = z[:, 1:2]
    m = jnp.maximum(z0, z1)
    lse = m + jnp.log(jnp.exp(z0 - m) + jnp.exp(z1 - m))
    o_ref[...] = z - lse


def _head(x, w1, b1, w2, b2, bm=256):
    grid = (MP // bm,)
    return pl.pallas_call(
        _head_body,
        grid=grid,
        in_specs=[
            pl.BlockSpec((bm, D2P), lambda mi: (mi, 0)),
            pl.BlockSpec((D2P, HW1), lambda mi: (0, 0)),
            pl.BlockSpec((8, HW1), lambda mi: (0, 0)),
            pl.BlockSpec((HW1, HW2), lambda mi: (0, 0)),
            pl.BlockSpec((8, HW2), lambda mi: (0, 0)),
        ],
        out_specs=pl.BlockSpec((bm, HW2), lambda mi: (mi, 0)),
        out_shape=jax.ShapeDtypeStruct((MP, HW2), jnp.float32),
    )(x, w1, b1, w2, b2)


def _edge_meta(dst_s):
    bstart = jnp.searchsorted(
        dst_s, jnp.arange(NBLK + 1, dtype=jnp.int32) * NBT).astype(jnp.int32)
    lot = bstart[:-1]
    hit = bstart[1:]
    lo16 = lot // 16 * 16
    nwin = (hit - lo16 + 127) // 128
    return jnp.concatenate([lo16, nwin, lot, hit]).astype(jnp.int32)


def kernel(x, edge_index, edge_type, gene_emb, W1, root1, b1, W2, root2, b2,
           lin1_w, lin1_b, lin2_w, lin2_b):
    f32 = jnp.float32

    h1 = jnp.zeros((MP, 1613), f32)
    h1 = h1.at[:N_X].set(x).at[N_X:N_TOTAL].set(gene_emb)

    root2p = jnp.zeros((D1, D2P), f32).at[:, :900].set(root2)
    b2p = jnp.zeros((D2P,), f32).at[:900].set(b2)

    l1w = jnp.zeros((D2P, HW1), f32).at[:900, :400].set(lin1_w)
    l1b = jnp.zeros((8, HW1), f32).at[0, :400].set(lin1_b)
    l2w = jnp.zeros((HW1, HW2), f32).at[:400, :2].set(lin2_w)
    l2b = jnp.zeros((8, HW2), f32).at[0, :2].set(lin2_b)

    src = edge_index[0].astype(jnp.int32)
    dst = edge_index[1].astype(jnp.int32)
    et = edge_type.astype(jnp.int32)
    order = jnp.argsort(dst)
    npad = E_PAD - E
    src_s = jnp.concatenate([src[order], jnp.zeros((npad,), jnp.int32)])
    dst_s = jnp.concatenate([dst[order], jnp.full((npad,), MP, jnp.int32)])
    et_s = jnp.concatenate([et[order], jnp.zeros((npad,), jnp.int32)])

    g_y = et_s * MP + src_s
    g_c = jnp.where(jnp.arange(E_PAD) < E,
                    et_s * N_TOTAL + dst_s, CNT_SLOTS - 1)
    g_c = g_c.astype(jnp.int32)

    meta = _edge_meta(dst_s)

    scale = _sc_scale(g_c)
    scale_bits = lax.bitcast_convert_type(scale, jnp.int32)
    eds = jnp.stack([dst_s.reshape(-1, 16),
                     scale_bits.reshape(-1, 16)], axis=1).reshape(-1)

    y1m = _mm_msg(h1, W1, D1P).reshape(4 * MP, D1P)
    y1r = _mm_root(h1, root1)
    h2 = _sc_msg(y1m, y1r, g_y, eds, meta, b1, D1P, D1 // 16, D1)

    y2m = _mm_msg(h2, W2, D2P).reshape(4 * MP, D2P)
    y2r = _mm_root(h2, root2p)
    h3 = _sc_msg(y2m, y2r, g_y, eds, meta, b2p, D2P, 57, D2P)

    out = _head(h3, l1w, l1b, l2w, l2b)
    return out[:N_TOTAL, :2]


# speedup vs baseline: 1.1020x; 1.0014x over previous
import functools

import jax
import jax.numpy as jnp
from jax import lax
from jax.experimental import pallas as pl
from jax.experimental.pallas import tpu as pltpu
from jax.experimental.pallas import tpu_sc as plsc

N_X = 5736
N_GENE = 4264
N_TOTAL = N_X + N_GENE
E = 48000
NUM_REL = 4

MP = 10240
K1P = 1664
D1 = 1600
D1P = 1664
D2P = 1024
HW1 = 512
HW2 = 128

E_PAD = 49152
CB = 32
CNT_SLOTS = 49152

_MESH = dict(core_axis_name="c", subcore_axis_name="s")


def _lane(vec, i):
    return jnp.sum(jnp.where(lax.iota(jnp.int32, 16) == i, vec, 0))


def _sc_scale_body(gc_hbm, scale_hbm, cnt_v, gblk_v, sblk_v):
    c = lax.axis_index("c")
    s = lax.axis_index("s")
    ones = jnp.ones((16,), jnp.float32)

    @pl.when(jnp.logical_and(c == 0, s == 0))
    def _():
        def zero_body(i, _):
            cnt_v[pl.ds(i * 16, 16)] = jnp.zeros((16,), jnp.float32)
            return 0
        lax.fori_loop(0, CNT_SLOTS // 16, zero_body, 0)

        def count_blk(blk, _):
            pltpu.sync_copy(gc_hbm.at[pl.ds(blk * 1024, 1024)], gblk_v)

            def count_in(k, _):
                idx = gblk_v[pl.ds(k * 16, 16)]
                plsc.addupdate_scatter(cnt_v, [idx], ones)
                return 0
            lax.fori_loop(0, 64, count_in, 0)
            return 0
        lax.fori_loop(0, E_PAD // 1024, count_blk, 0)

        def scale_blk(blk, _):
            pltpu.sync_copy(gc_hbm.at[pl.ds(blk * 1024, 1024)], gblk_v)

            def scale_in(k, _):
                idx = gblk_v[pl.ds(k * 16, 16)]
                vals = plsc.load_gather(cnt_v, [idx])
                sblk_v[pl.ds(k * 16, 16)] = 1.0 / jnp.maximum(vals, 1.0)
                return 0
            lax.fori_loop(0, 64, scale_in, 0)
            pltpu.sync_copy(sblk_v, scale_hbm.at[pl.ds(blk * 1024, 1024)])
            return 0
        lax.fori_loop(0, E_PAD // 1024, scale_blk, 0)


def _sc_scale(g_cnt):
    kern = pl.kernel(
        _sc_scale_body,
        out_type=jax.ShapeDtypeStruct((E_PAD,), jnp.float32),
        mesh=plsc.VectorSubcoreMesh(**_MESH),
        compiler_params=pltpu.CompilerParams(needs_layout_passes=False),
        scratch_types=[
            pltpu.VMEM((CNT_SLOTS,), jnp.float32),
            pltpu.VMEM((1024,), jnp.int32),
            pltpu.VMEM((1024,), jnp.float32),
        ],
    )
    return kern(g_cnt)


NBT = 32
_EORDER = [0, 4, 8, 12, 1, 5, 9, 13, 2, 6, 10, 14, 3, 7, 11, 15]
NBLK = MP // NBT


def _sc_msg_body(dpad, nja, dw, ym_hbm, yr_hbm, g_hbm, eds_hbm, meta_hbm, bias_hbm, out_hbm,
                 meta_v, bias_v, eds_v, idxa_v, idxb_v, rows_v, acc_v,
                 y8_v, gsem0, gsem1):
    c = lax.axis_index("c")
    s = lax.axis_index("s")
    w = c * 16 + s
    iota = lax.iota(jnp.int32, 16)
    nj = dpad // 16
    njw = dw // 16

    pltpu.sync_copy(meta_hbm, meta_v)
    pltpu.sync_copy(bias_hbm, bias_v)

    def z0(i, _):
        acc_v[pl.ds(i * 16, 16)] = jnp.zeros((16,), jnp.float32)
        return 0
    lax.fori_loop(0, (NBT + 1) * nj, z0, 0)

    gsems = (gsem0, gsem1)
    idxs = (idxa_v, idxb_v)

    def block_body(k, _):
        bid = k * 32 + w
        cb = pl.multiple_of(bid // 16 * 16, 16)
        ln = bid - cb
        lo8 = _lane(meta_v[pl.ds(cb, 16)], ln)
        nwin = _lane(meta_v[pl.ds(NBLK + cb, 16)], ln)
        lot = _lane(meta_v[pl.ds(2 * NBLK + cb, 16)], ln)
        hit = _lane(meta_v[pl.ds(3 * NBLK + cb, 16)], ln)
        bnb = bid * NBT

        def win_body(wi, _):
            wbase = pl.multiple_of(lo8 + wi * 128, 8)
            eoff = pl.multiple_of(wbase * 2, 8)
            pltpu.sync_copy(eds_hbm.at[pl.ds(eoff, 256)], eds_v)

            def issue(sub, par):
                sb8 = pl.multiple_of(wbase + sub * 16, 8)
                pltpu.sync_copy(g_hbm.at[pl.ds(sb8, 16)], idxs[par])
                pltpu.async_copy(ym_hbm.at[idxs[par]],
                                 rows_v.at[par], gsems[par])

            def wait(sub, par):
                pltpu.make_async_copy(ym_hbm.at[idxs[par]],
                                      rows_v.at[par], gsems[par]).wait()

            def process(sub, par):
                sbase = wbase + sub * 16
                evec = sbase + iota
                dl = eds_v[pl.ds(sub * 32, 16)] - bnb
                inr = ((evec >= lot) & (evec < hit)
                       & (dl >= 0) & (dl < NBT))
                dlc = jnp.where(inr, dl, NBT)
                sv = jnp.where(
                    inr, plsc.bitcast(eds_v[pl.ds(sub * 32 + 16, 16)],
                                      jnp.float32),
                    jnp.zeros((16,), jnp.float32))
                sscs = [sv[e] for e in range(16)]
                abss = [dlc[e] * dpad for e in range(16)]

                def jcol(j, _, par=par):
                    off = j * 16
                    for e in _EORDER:
                        plsc.addupdate(
                            acc_v.at[pl.ds(abss[e] + off, 16)],
                            rows_v[par, e, pl.ds(off, 16)] * sscs[e])
                    return 0
                lax.fori_loop(0, nj, jcol, 0)

            @pl.when(wbase < hit)
            def _():
                issue(0, 0)
            for sub in range(8):
                par = sub & 1
                if sub + 1 < 8:
                    @pl.when(wbase + (sub + 1) * 16 < hit)
                    def _(sub=sub, par=par):
                        issue(sub + 1, 1 - par)

                @pl.when(wbase + sub * 16 < hit)
                def _(sub=sub, par=par):
                    wait(sub, par)
                    process(sub, par)
            return 0
        lax.fori_loop(0, nwin, win_body, 0)

        def drain_body(r8, _):
            grow = pl.multiple_of(bnb + r8 * 8, 8)
            pltpu.sync_copy(yr_hbm.at[pl.ds(grow, 8)], y8_v)
            for rr in range(8):
                def db(j, _, rr=rr):
                    sl = pl.ds(j * 16, 16)
                    aoff = pl.ds((r8 * 8 + rr) * dpad + j * 16, 16)
                    y8_v[rr, sl] = jnp.maximum(
                        acc_v[aoff] + y8_v[rr, sl] + bias_v[sl], 0.0)
                    acc_v[aoff] = jnp.zeros((16,), jnp.float32)
                    return 0
                lax.fori_loop(0, njw, db, 0, unroll=4)
            pltpu.sync_copy(y8_v, out_hbm.at[pl.ds(grow, 8)])
            return 0
        lax.fori_loop(0, NBT // 8, drain_body, 0)
        return 0
    lax.fori_loop(0, NBLK // 32, block_body, 0)


def _sc_msg(ymsg, yroot, g_y, eds, meta, bias, dpad, nja, dw):
    kern = pl.kernel(
        functools.partial(_sc_msg_body, dpad, nja, dw),
        out_type=jax.ShapeDtypeStruct((MP, dw), jnp.float32),
        mesh=plsc.VectorSubcoreMesh(**_MESH),
        compiler_params=pltpu.CompilerParams(needs_layout_passes=False),
        scratch_types=[
            pltpu.VMEM((4 * NBLK,), jnp.int32),
            pltpu.VMEM((dw,), jnp.float32),
            pltpu.VMEM((256,), jnp.int32),
            pltpu.VMEM((16,), jnp.int32),
            pltpu.VMEM((16,), jnp.int32),
            pltpu.VMEM((2, 16, dpad), jnp.float32),
            pltpu.VMEM(((NBT + 1) * dpad,), jnp.float32),
            pltpu.VMEM((8, dw), jnp.float32),
            pltpu.SemaphoreType.DMA,
            pltpu.SemaphoreType.DMA,
        ],
    )
    return kern(ymsg, yroot, g_y, eds, meta, bias)


def _mm_msg_body(dpad, x_ref, w_ref, o_ref):
    d = jnp.dot(x_ref[...], w_ref[0], preferred_element_type=jnp.float32)
    pad = dpad - d.shape[1]
    if pad:
        d = jnp.concatenate(
            [d, jnp.zeros((d.shape[0], pad), jnp.float32)], axis=1)
    o_ref[0] = d


def _mm_msg(x, w, dpad, bm=256):
    r, k, dout = w.shape
    return pl.pallas_call(
        functools.partial(_mm_msg_body, dpad),
        grid=(r, MP // bm),
        in_specs=[
            pl.BlockSpec((bm, k), lambda ri, mi: (mi, 0)),
            pl.BlockSpec((1, k, dout), lambda ri, mi: (ri, 0, 0)),
        ],
        out_specs=pl.BlockSpec((1, bm, dpad), lambda ri, mi: (ri, mi, 0)),
        out_shape=jax.ShapeDtypeStruct((r, MP, dpad), jnp.float32),
    )(x, w)


def _mm_root_body(x_ref, w_ref, o_ref):
    o_ref[...] = jnp.dot(x_ref[...], w_ref[...],
                         preferred_element_type=jnp.float32)


def _mm_root(x, w, bm=256):
    k, dout = w.shape
    return pl.pallas_call(
        _mm_root_body,
        grid=(MP // bm,),
        in_specs=[
            pl.BlockSpec((bm, k), lambda mi: (mi, 0)),
            pl.BlockSpec((k, dout), lambda mi: (0, 0)),
        ],
        out_specs=pl.BlockSpec((bm, dout), lambda mi: (mi, 0)),
        out_shape=jax.ShapeDtypeStruct((MP, dout), jnp.float32),
    )(x, w)


def _head_body(x_ref, w1_ref, b1_ref, w2_ref, b2_ref, o_ref):
    a = jnp.dot(x_ref[...], w1_ref[...], preferred_element_type=jnp.float32)
    a = jnp.maximum(a + b1_ref[0:1, :], 0.0)
    z = jnp.dot(a, w2_ref[...], preferred_element_type=jnp.float32)
    z = z + b2_ref[0:1, :]
    z0 = z[:, 0:1]
    z1 = z[:, 1:2]
    m = jnp.maximum(z0, z1)
    lse = m + jnp.log(jnp.exp(z0 - m) + jnp.exp(z1 - m))
    o_ref[...] = z - lse


def _head(x, w1, b1, w2, b2, bm=256):
    grid = (MP // bm,)
    return pl.pallas_call(
        _head_body,
        grid=grid,
        in_specs=[
            pl.BlockSpec((bm, D2P), lambda mi: (mi, 0)),
            pl.BlockSpec((D2P, HW1), lambda mi: (0, 0)),
            pl.BlockSpec((8, HW1), lambda mi: (0, 0)),
            pl.BlockSpec((HW1, HW2), lambda mi: (0, 0)),
            pl.BlockSpec((8, HW2), lambda mi: (0, 0)),
        ],
        out_specs=pl.BlockSpec((bm, HW2), lambda mi: (mi, 0)),
        out_shape=jax.ShapeDtypeStruct((MP, HW2), jnp.float32),
    )(x, w1, b1, w2, b2)


def _edge_meta(dst_s):
    bstart = jnp.searchsorted(
        dst_s, jnp.arange(NBLK + 1, dtype=jnp.int32) * NBT).astype(jnp.int32)
    lot = bstart[:-1]
    hit = bstart[1:]
    lo16 = lot // 16 * 16
    nwin = (hit - lo16 + 127) // 128
    return jnp.concatenate([lo16, nwin, lot, hit]).astype(jnp.int32)


def kernel(x, edge_index, edge_type, gene_emb, W1, root1, b1, W2, root2, b2,
           lin1_w, lin1_b, lin2_w, lin2_b):
    f32 = jnp.float32

    h1 = jnp.zeros((MP, 1613), f32)
    h1 = h1.at[:N_X].set(x).at[N_X:N_TOTAL].set(gene_emb)

    root2p = jnp.zeros((D1, D2P), f32).at[:, :900].set(root2)
    b2p = jnp.zeros((D2P,), f32).at[:900].set(b2)

    l1w = jnp.zeros((D2P, HW1), f32).at[:900, :400].set(lin1_w)
    l1b = jnp.zeros((8, HW1), f32).at[0, :400].set(lin1_b)
    l2w = jnp.zeros((HW1, HW2), f32).at[:400, :2].set(lin2_w)
    l2b = jnp.zeros((8, HW2), f32).at[0, :2].set(lin2_b)

    src = edge_index[0].astype(jnp.int32)
    dst = edge_index[1].astype(jnp.int32)
    et = edge_type.astype(jnp.int32)
    order = jnp.argsort(dst)
    npad = E_PAD - E
    src_s = jnp.concatenate([src[order], jnp.zeros((npad,), jnp.int32)])
    dst_s = jnp.concatenate([dst[order], jnp.full((npad,), MP, jnp.int32)])
    et_s = jnp.concatenate([et[order], jnp.zeros((npad,), jnp.int32)])

    g_y = et_s * MP + src_s
    g_c = jnp.where(jnp.arange(E_PAD) < E,
                    et_s * N_TOTAL + dst_s, CNT_SLOTS - 1)
    g_c = g_c.astype(jnp.int32)

    meta = _edge_meta(dst_s)

    scale = _sc_scale(g_c)
    scale_bits = lax.bitcast_convert_type(scale, jnp.int32)
    eds = jnp.stack([dst_s.reshape(-1, 16),
                     scale_bits.reshape(-1, 16)], axis=1).reshape(-1)

    y1m = _mm_msg(h1, W1, D1P).reshape(4 * MP, D1P)
    y1r = _mm_root(h1, root1)
    h2 = _sc_msg(y1m, y1r, g_y, eds, meta, b1, D1P, D1 // 16, D1)

    y2m = _mm_msg(h2, W2, D2P).reshape(4 * MP, D2P)
    y2r = _mm_root(h2, root2p)
    h3 = _sc_msg(y2m, y2r, g_y, eds, meta, b2p, D2P, 57, D2P)

    out = _head(h3, l1w, l1b, l2w, l2b)
    return out[:N_TOTAL, :2]
